# Initial kernel scaffold; baseline (speedup 1.0000x reference)
import jax
import jax.numpy as jnp
from jax.experimental import pallas as pl


def kernel(x, E_adaptive, fc_w, fc_b):
    raise NotImplementedError("write your pallas kernel here")


# speedup vs baseline: 150.2522x; 150.2522x over previous
import math

import jax
import jax.numpy as jnp
from jax.experimental import pallas as pl


def _adp_kernel(e_ref, out_ref):
    E = e_ref[...].astype(jnp.bfloat16)
    g = jax.lax.dot_general(E, E, (((1,), (1,)), ((), ())),
                            preferred_element_type=jnp.float32)
    g = jnp.maximum(g, 0.0)
    m = jnp.max(g, axis=-1, keepdims=True)
    e = jnp.exp(g - m)
    s = jnp.sum(e, axis=-1, keepdims=True)
    out_ref[...] = e / s


def _main_kernel(xt_ref, a2_ref, fcw_ref, fcb_ref, out_ref, *, n_keep,
                 inv_sqrt_c):
    xs = jnp.sum(xt_ref[0], axis=0).astype(jnp.bfloat16)
    raw = jax.lax.dot_general(xs, xs, (((0,), (0,)), ((), ())),
                              preferred_element_type=jnp.float32)
    logits = jnp.maximum(raw * inv_sqrt_c, 0.0)
    m1 = jnp.max(logits, axis=-1, keepdims=True)
    e1 = jnp.exp(logits - m1)
    s1 = jnp.sum(e1, axis=-1, keepdims=True)
    dyn = e1 / s1
    dyn_b = dyn.astype(jnp.bfloat16).astype(jnp.float32)
    adp_b = a2_ref[...].astype(jnp.bfloat16).astype(jnp.float32)
    w0 = fcw_ref[0, 0].astype(jnp.bfloat16).astype(jnp.float32)
    w1 = fcw_ref[0, 1].astype(jnp.bfloat16).astype(jnp.float32)
    fused = (dyn_b * w0 + adp_b * w1) + fcb_ref[0, 0]
    m2 = jnp.max(fused, axis=-1, keepdims=True)
    e2 = jnp.exp(fused - m2)
    s2 = jnp.sum(e2, axis=-1, keepdims=True)
    adj_f = e2 / s2

    keys = jax.lax.bitcast_convert_type(adj_f, jnp.int32)
    n = keys.shape[0]
    kf = jnp.float32(n_keep)

    def body(i, prefix):
        cand = prefix | (jnp.int32(1) << (jnp.int32(29) - i))
        cnt = jnp.sum(jnp.where(keys >= cand, 1.0, 0.0), axis=-1,
                      keepdims=True)
        return jnp.where(cnt >= kf, cand, prefix)

    thresh = jax.lax.fori_loop(0, 30, body, jnp.zeros((n, 1), jnp.int32))

    cnt_gt = jnp.sum(jnp.where(keys > thresh, 1.0, 0.0), axis=-1,
                     keepdims=True)
    r = kf - cnt_gt
    kf2 = jnp.float32(n) - r + 1.0
    eq = keys == thresh
    idx = jax.lax.broadcasted_iota(jnp.int32, keys.shape, 1)
    mvals = jnp.where(eq, idx, jnp.int32(n))
    idx_bits = int(n).bit_length()

    def body2(i, prefix):
        cand = prefix | (jnp.int32(1) << (jnp.int32(idx_bits - 1) - i))
        cnt = jnp.sum(jnp.where(mvals >= cand, 1.0, 0.0), axis=-1,
                      keepdims=True)
        return jnp.where(cnt >= kf2, cand, prefix)

    last_idx = jax.lax.fori_loop(0, idx_bits, body2,
                                 jnp.zeros((n, 1), jnp.int32))
    keep = (keys > thresh) | (eq & (idx <= last_idx))
    out_ref[0] = jnp.where(keep, adj_f, 0.0)


def kernel(x, E_adaptive, fc_w, fc_b):
    B, C, N, T = x.shape
    n_keep = int(N * 0.8)
    fc_b2 = fc_b.reshape(1, 1)

    a2 = pl.pallas_call(
        _adp_kernel,
        out_shape=jax.ShapeDtypeStruct((N, N), jnp.float32),
        in_specs=[
            pl.BlockSpec(E_adaptive.shape, lambda: (0, 0)),
        ],
        out_specs=pl.BlockSpec((N, N), lambda: (0, 0)),
    )(E_adaptive)

    x_t = jnp.transpose(x, (0, 3, 1, 2))

    import functools
    body = functools.partial(_main_kernel, n_keep=n_keep,
                             inv_sqrt_c=float(1.0 / math.sqrt(C)))
    out = pl.pallas_call(
        body,
        grid=(B,),
        out_shape=jax.ShapeDtypeStruct((B, N, N), jnp.float32),
        in_specs=[
            pl.BlockSpec((1, T, C, N), lambda b: (b, 0, 0, 0)),
            pl.BlockSpec((N, N), lambda b: (0, 0)),
            pl.BlockSpec((1, 2), lambda b: (0, 0)),
            pl.BlockSpec((1, 1), lambda b: (0, 0)),
        ],
        out_specs=pl.BlockSpec((1, N, N), lambda b: (b, 0, 0)),
    )(x_t, a2, fc_w, fc_b2)
    return out
